# baseline (speedup 1.0000x reference)
import functools

import jax
import jax.numpy as jnp
from jax import lax
from jax.experimental import pallas as pl
from jax.experimental.pallas import tpu as pltpu
from jax.experimental.pallas import tpu_sc as plsc

_T, _D, _E, _K, _DFF, _OUT, _CAP = 2048, 1024, 8, 2, 2048, 1024, 640
_TRASH = _E * _CAP
_EIN_ROWS = _E * _CAP + 8
_TB = 256
_NC, _NS = 2, 16
_NW = _NC * _NS
_SLOTS = _T * _K
_PER_W = _SLOTS // _NW
_CHUNK = 32
_NCHUNK = _PER_W // _CHUNK


def _routing_math(lg, d0_ref, d1_ref, gi0_ref, gi1_ref, gk0_ref, gk1_ref):
    lanes = lax.broadcasted_iota(jnp.int32, (_T, _E), 1)
    m0 = jnp.max(lg, axis=1, keepdims=True)
    i0 = jnp.min(jnp.where(lg == m0, lanes, _E), axis=1, keepdims=True)
    masked = jnp.where(lanes == i0, -jnp.inf, lg)
    m1 = jnp.max(masked, axis=1, keepdims=True)
    i1 = jnp.min(jnp.where(masked == m1, lanes, _E), axis=1, keepdims=True)
    e1m = jnp.exp(m1 - m0)
    denom = 1.0 + e1m
    g0 = 1.0 / denom
    g1 = e1m / denom
    oh0 = (lanes == i0).astype(jnp.float32)
    oh1 = (lanes == i1).astype(jnp.float32)
    cnt = (oh0 + oh1).astype(jnp.bfloat16)
    row = lax.broadcasted_iota(jnp.int32, (_T, _T), 0)
    col = lax.broadcasted_iota(jnp.int32, (_T, _T), 1)
    tril = (col < row).astype(jnp.bfloat16)
    csum = jnp.dot(tril, cnt, preferred_element_type=jnp.float32)
    pos0 = jnp.sum(csum * oh0, axis=1, keepdims=True).astype(jnp.int32)
    pos1 = jnp.sum(csum * oh1, axis=1, keepdims=True).astype(jnp.int32)
    keep0 = pos0 < _CAP
    keep1 = pos1 < _CAP
    d0_ref[...] = jnp.where(keep0, i0 * _CAP + pos0, _TRASH)
    d1_ref[...] = jnp.where(keep1, i1 * _CAP + pos1, _TRASH)
    gi0_ref[...] = i0 * _CAP + jnp.minimum(pos0, _CAP - 1)
    gi1_ref[...] = i1 * _CAP + jnp.minimum(pos1, _CAP - 1)
    gk0_ref[...] = jnp.where(keep0, g0, 0.0)
    gk1_ref[...] = jnp.where(keep1, g1, 0.0)


def _projroute_body(x_ref, w_ref, b_ref, wr_ref,
                    h_ref, d0_ref, d1_ref, gi0_ref, gi1_ref, gk0_ref, gk1_ref,
                    lg_acc):
    i = pl.program_id(0)
    h = jnp.dot(x_ref[...], w_ref[...], preferred_element_type=jnp.float32)
    h = h + b_ref[...]
    h_ref[...] = h
    lg_acc[pl.ds(i * _TB, _TB), :] = jnp.dot(
        h, wr_ref[...], preferred_element_type=jnp.float32)

    @pl.when(i == _T // _TB - 1)
    def _():
        _routing_math(lg_acc[...], d0_ref, d1_ref, gi0_ref, gi1_ref,
                      gk0_ref, gk1_ref)


def _projroute(x, W_in, b_in2d, W_router):
    col_i = pl.BlockSpec((_T, 1), lambda i: (0, 0))
    return pl.pallas_call(
        _projroute_body,
        grid=(_T // _TB,),
        in_specs=[
            pl.BlockSpec((_TB, _D), lambda i: (i, 0)),
            pl.BlockSpec((_D, _D), lambda i: (0, 0)),
            pl.BlockSpec((1, _D), lambda i: (0, 0)),
            pl.BlockSpec((_D, _E), lambda i: (0, 0)),
        ],
        out_specs=[pl.BlockSpec((_TB, _D), lambda i: (i, 0))] + [col_i] * 6,
        out_shape=[jax.ShapeDtypeStruct((_T, _D), jnp.float32)] + [
            jax.ShapeDtypeStruct((_T, 1), jnp.int32),
            jax.ShapeDtypeStruct((_T, 1), jnp.int32),
            jax.ShapeDtypeStruct((_T, 1), jnp.int32),
            jax.ShapeDtypeStruct((_T, 1), jnp.int32),
            jax.ShapeDtypeStruct((_T, 1), jnp.float32),
            jax.ShapeDtypeStruct((_T, 1), jnp.float32),
        ],
        scratch_shapes=[pltpu.VMEM((_T, _E), jnp.float32)],
    )(x, W_in, b_in2d, W_router)


def _dispatch(h, scat_idx):
    mesh = plsc.VectorSubcoreMesh(core_axis_name="c", subcore_axis_name="s")

    @functools.partial(
        pl.kernel,
        out_type=jax.ShapeDtypeStruct((_EIN_ROWS, _D), jnp.float32),
        mesh=mesh,
        scratch_types=[
            pltpu.VMEM((_CHUNK,), jnp.int32),
            pltpu.VMEM((_CHUNK,), jnp.int32),
            pltpu.VMEM((_CHUNK, _D), jnp.float32),
            pltpu.VMEM((_CHUNK, _D), jnp.float32),
            pltpu.SemaphoreType.DMA,
            pltpu.SemaphoreType.DMA,
        ],
    )
    def k(h_hbm, idx_hbm, out_hbm, i0, i1, r0, r1, s0, s1):
        wid = lax.axis_index("s") * _NC + lax.axis_index("c")
        base = wid * _PER_W
        idx_v, rows_v, sem = [i0, i1], [r0, r1], [s0, s1]
        cps = [None, None]
        for ci in range(_NCHUNK):
            b = ci % 2
            if cps[b] is not None:
                cps[b].wait()
            off = base + ci * _CHUNK
            tok = lax.rem(off, _T)
            pltpu.sync_copy(idx_hbm.at[pl.ds(off, _CHUNK)], idx_v[b])
            pltpu.sync_copy(h_hbm.at[pl.ds(tok, _CHUNK)], rows_v[b])
            cps[b] = pltpu.async_copy(rows_v[b], out_hbm.at[idx_v[b]], sem[b])
        cps[0].wait()
        cps[1].wait()

    return k(h, scat_idx)


def _ffn_body(xin_ref, w1_ref, b1_ref, w2_ref, b2_ref, eo_ref):
    x = xin_ref[...].astype(jnp.bfloat16)
    w1 = w1_ref[0].astype(jnp.bfloat16)
    hmid = jnp.dot(x, w1, preferred_element_type=jnp.float32)
    hmid = jnp.maximum(hmid + b1_ref[0], 0.0).astype(jnp.bfloat16)
    w2 = w2_ref[0].astype(jnp.bfloat16)
    eo = jnp.dot(hmid, w2, preferred_element_type=jnp.float32)
    eo_ref[...] = eo + b2_ref[0]


def _ffn(ei, W1, b1, W2, b2):
    return pl.pallas_call(
        _ffn_body,
        grid=(_E,),
        in_specs=[
            pl.BlockSpec((_CAP, _D), lambda e: (e, 0)),
            pl.BlockSpec((1, _D, _DFF), lambda e: (e, 0, 0)),
            pl.BlockSpec((1, 1, _DFF), lambda e: (e, 0, 0)),
            pl.BlockSpec((1, _DFF, _D), lambda e: (e, 0, 0)),
            pl.BlockSpec((1, 1, _D), lambda e: (e, 0, 0)),
        ],
        out_specs=pl.BlockSpec((_CAP, _D), lambda e: (e, 0)),
        out_shape=jax.ShapeDtypeStruct((_E * _CAP, _D), jnp.float32),
    )(ei, W1, b1.reshape(_E, 1, _DFF), W2, b2.reshape(_E, 1, _D))


def _combine(eo, gidx):
    mesh = plsc.VectorSubcoreMesh(core_axis_name="c", subcore_axis_name="s")

    @functools.partial(
        pl.kernel,
        out_type=jax.ShapeDtypeStruct((_SLOTS, _D), jnp.float32),
        mesh=mesh,
        scratch_types=[
            pltpu.VMEM((_CHUNK,), jnp.int32),
            pltpu.VMEM((_CHUNK,), jnp.int32),
            pltpu.VMEM((_CHUNK, _D), jnp.float32),
            pltpu.VMEM((_CHUNK, _D), jnp.float32),
            pltpu.SemaphoreType.DMA,
            pltpu.SemaphoreType.DMA,
        ],
    )
    def k(eo_hbm, idx_hbm, out_hbm, i0, i1, r0, r1, s0, s1):
        wid = lax.axis_index("s") * _NC + lax.axis_index("c")
        base = wid * _PER_W
        idx_v, rows_v, sem = [i0, i1], [r0, r1], [s0, s1]
        cps = [None, None]
        for ci in range(_NCHUNK):
            b = ci % 2
            off = base + ci * _CHUNK
            if cps[b] is not None:
                cps[b].wait()
                prev = base + (ci - 2) * _CHUNK
                pltpu.sync_copy(rows_v[b], out_hbm.at[pl.ds(prev, _CHUNK)])
            pltpu.sync_copy(idx_hbm.at[pl.ds(off, _CHUNK)], idx_v[b])
            cps[b] = pltpu.async_copy(eo_hbm.at[idx_v[b]], rows_v[b], sem[b])
        for b in range(2):
            cps[b].wait()
            prev = base + (_NCHUNK - 2 + b) * _CHUNK
            pltpu.sync_copy(rows_v[b], out_hbm.at[pl.ds(prev, _CHUNK)])

    return k(eo, gidx)


def _head_body(g0_ref, g1_ref, gk0_ref, gk1_ref, wh_ref, o_ref):
    gk0 = gk0_ref[...]
    gk1 = gk1_ref[...]
    moe = jnp.where(gk0 > 0.0, g0_ref[...] * gk0, 0.0)
    moe = moe + jnp.where(gk1 > 0.0, g1_ref[...] * gk1, 0.0)
    o_ref[...] = jnp.dot(moe.astype(jnp.bfloat16),
                         wh_ref[...].astype(jnp.bfloat16),
                         preferred_element_type=jnp.float32)


def _head(g, gk0, gk1, W_head):
    return pl.pallas_call(
        _head_body,
        grid=(_T // _TB,),
        in_specs=[
            pl.BlockSpec((_TB, _D), lambda i: (i, 0)),
            pl.BlockSpec((_TB, _D), lambda i: (i + _T // _TB, 0)),
            pl.BlockSpec((_TB, 1), lambda i: (i, 0)),
            pl.BlockSpec((_TB, 1), lambda i: (i, 0)),
            pl.BlockSpec((_D, _OUT), lambda i: (0, 0)),
        ],
        out_specs=pl.BlockSpec((_TB, _OUT), lambda i: (i, 0)),
        out_shape=jax.ShapeDtypeStruct((_T, _OUT), jnp.float32),
    )(g, g, gk0, gk1, W_head)


def kernel(x, W_in, b_in, W_router, W1, b1, W2, b2, W_head):
    h, d0, d1, gi0, gi1, gk0, gk1 = _projroute(
        x, W_in, b_in.reshape(1, _D), W_router)
    scat = jnp.concatenate([d0.reshape(_T), d1.reshape(_T)])
    gidx = jnp.concatenate([gi0.reshape(_T), gi1.reshape(_T)])
    ei = _dispatch(h, scat)
    eo = _ffn(ei, W1, b1, W2, b2)
    g = _combine(eo, gidx)
    return _head(g, gk0, gk1, W_head)


# speedup vs baseline: 1.1484x; 1.1484x over previous
import functools

import jax
import jax.numpy as jnp
from jax import lax
from jax.experimental import pallas as pl
from jax.experimental.pallas import tpu as pltpu
from jax.experimental.pallas import tpu_sc as plsc

_T, _D, _E, _K, _DFF, _OUT, _CAP = 2048, 1024, 8, 2, 2048, 1024, 640
_TRASH = _E * _CAP
_EIN_ROWS = _E * _CAP + 8
_TB = 256
_NC, _NS = 2, 16
_NW = _NC * _NS
_SLOTS = _T * _K
_PER_W = _SLOTS // _NW
_CHUNK = 64
_NCHUNK = _PER_W // _CHUNK
_D2 = _D // 2


def _routing_math(lg, d0_ref, d1_ref, gi0_ref, gi1_ref, gk0_ref, gk1_ref):
    lanes = lax.broadcasted_iota(jnp.int32, (_T, _E), 1)
    m0 = jnp.max(lg, axis=1, keepdims=True)
    i0 = jnp.min(jnp.where(lg == m0, lanes, _E), axis=1, keepdims=True)
    masked = jnp.where(lanes == i0, -jnp.inf, lg)
    m1 = jnp.max(masked, axis=1, keepdims=True)
    i1 = jnp.min(jnp.where(masked == m1, lanes, _E), axis=1, keepdims=True)
    e1m = jnp.exp(m1 - m0)
    denom = 1.0 + e1m
    g0 = 1.0 / denom
    g1 = e1m / denom
    oh0 = (lanes == i0).astype(jnp.float32)
    oh1 = (lanes == i1).astype(jnp.float32)
    cnt = (oh0 + oh1).astype(jnp.bfloat16)
    row = lax.broadcasted_iota(jnp.int32, (_T, _T), 0)
    col = lax.broadcasted_iota(jnp.int32, (_T, _T), 1)
    tril = (col < row).astype(jnp.bfloat16)
    csum = jnp.dot(tril, cnt, preferred_element_type=jnp.float32)
    pos0 = jnp.sum(csum * oh0, axis=1, keepdims=True).astype(jnp.int32)
    pos1 = jnp.sum(csum * oh1, axis=1, keepdims=True).astype(jnp.int32)
    keep0 = pos0 < _CAP
    keep1 = pos1 < _CAP
    d0_ref[...] = jnp.where(keep0, i0 * _CAP + pos0, _TRASH)
    d1_ref[...] = jnp.where(keep1, i1 * _CAP + pos1, _TRASH)
    gi0_ref[...] = i0 * _CAP + jnp.minimum(pos0, _CAP - 1)
    gi1_ref[...] = i1 * _CAP + jnp.minimum(pos1, _CAP - 1)
    gk0_ref[...] = jnp.where(keep0, g0, 0.0)
    gk1_ref[...] = jnp.where(keep1, g1, 0.0)


def _projroute_body(x_ref, w_ref, b_ref, wr_ref,
                    h_ref, d0_ref, d1_ref, gi0_ref, gi1_ref, gk0_ref, gk1_ref,
                    lg_acc):
    i = pl.program_id(0)
    h = jnp.dot(x_ref[...], w_ref[...], preferred_element_type=jnp.float32)
    h = h + b_ref[...]
    h_ref[...] = pltpu.pack_elementwise(
        [h[:, :_D2], h[:, _D2:]], packed_dtype=jnp.bfloat16)
    lg_acc[pl.ds(i * _TB, _TB), :] = jnp.dot(
        h, wr_ref[...], preferred_element_type=jnp.float32)

    @pl.when(i == _T // _TB - 1)
    def _():
        _routing_math(lg_acc[...], d0_ref, d1_ref, gi0_ref, gi1_ref,
                      gk0_ref, gk1_ref)


def _projroute(x, W_in, b_in2d, W_router):
    col_i = pl.BlockSpec((_T, 1), lambda i: (0, 0))
    return pl.pallas_call(
        _projroute_body,
        grid=(_T // _TB,),
        in_specs=[
            pl.BlockSpec((_TB, _D), lambda i: (i, 0)),
            pl.BlockSpec((_D, _D), lambda i: (0, 0)),
            pl.BlockSpec((1, _D), lambda i: (0, 0)),
            pl.BlockSpec((_D, _E), lambda i: (0, 0)),
        ],
        out_specs=[pl.BlockSpec((_TB, _D2), lambda i: (i, 0))] + [col_i] * 6,
        out_shape=[jax.ShapeDtypeStruct((_T, _D2), jnp.int32)] + [
            jax.ShapeDtypeStruct((_T, 1), jnp.int32),
            jax.ShapeDtypeStruct((_T, 1), jnp.int32),
            jax.ShapeDtypeStruct((_T, 1), jnp.int32),
            jax.ShapeDtypeStruct((_T, 1), jnp.int32),
            jax.ShapeDtypeStruct((_T, 1), jnp.float32),
            jax.ShapeDtypeStruct((_T, 1), jnp.float32),
        ],
        scratch_shapes=[pltpu.VMEM((_T, _E), jnp.float32)],
    )(x, W_in, b_in2d, W_router)


def _dispatch(h, scat_idx):
    mesh = plsc.VectorSubcoreMesh(core_axis_name="c", subcore_axis_name="s")

    @functools.partial(
        pl.kernel,
        out_type=jax.ShapeDtypeStruct((_EIN_ROWS, _D2), jnp.int32),
        mesh=mesh,
        scratch_types=[
            pltpu.VMEM((_CHUNK,), jnp.int32),
            pltpu.VMEM((_CHUNK, _D2), jnp.int32),
            pltpu.SemaphoreType.DMA,
        ],
    )
    def k(h_hbm, idx_hbm, out_hbm, idx_v, rows_v, sem):
        wid = lax.axis_index("s") * _NC + lax.axis_index("c")
        base = wid * _PER_W

        @pl.loop(0, _NCHUNK)
        def _(ci):
            off = base + ci * _CHUNK
            tok = lax.rem(off, _T)
            pltpu.sync_copy(idx_hbm.at[pl.ds(off, _CHUNK)], idx_v)
            pltpu.sync_copy(h_hbm.at[pl.ds(tok, _CHUNK)], rows_v)
            pltpu.async_copy(rows_v, out_hbm.at[idx_v], sem).wait()

    return k(h, scat_idx)


def _ffn_body(xin_ref, w1_ref, b1_ref, w2_ref, b2_ref, eo_ref):
    x32 = xin_ref[...]
    lo = pltpu.unpack_elementwise(
        x32, index=0, packed_dtype=jnp.bfloat16,
        unpacked_dtype=jnp.float32).astype(jnp.bfloat16)
    hi = pltpu.unpack_elementwise(
        x32, index=1, packed_dtype=jnp.bfloat16,
        unpacked_dtype=jnp.float32).astype(jnp.bfloat16)
    w1 = w1_ref[0]
    hmid = jnp.dot(lo, w1[:_D2].astype(jnp.bfloat16),
                   preferred_element_type=jnp.float32)
    hmid += jnp.dot(hi, w1[_D2:].astype(jnp.bfloat16),
                    preferred_element_type=jnp.float32)
    hmid = jnp.maximum(hmid + b1_ref[0], 0.0).astype(jnp.bfloat16)
    w2 = w2_ref[0].astype(jnp.bfloat16)
    eo = jnp.dot(hmid, w2, preferred_element_type=jnp.float32)
    eo = eo + b2_ref[0]
    eo_ref[...] = pltpu.pack_elementwise(
        [eo[:, :_D2], eo[:, _D2:]], packed_dtype=jnp.bfloat16)


def _ffn(ei, W1, b1, W2, b2):
    return pl.pallas_call(
        _ffn_body,
        grid=(_E,),
        in_specs=[
            pl.BlockSpec((_CAP, _D2), lambda e: (e, 0)),
            pl.BlockSpec((1, _D, _DFF), lambda e: (e, 0, 0)),
            pl.BlockSpec((1, 1, _DFF), lambda e: (e, 0, 0)),
            pl.BlockSpec((1, _DFF, _D), lambda e: (e, 0, 0)),
            pl.BlockSpec((1, 1, _D), lambda e: (e, 0, 0)),
        ],
        out_specs=pl.BlockSpec((_CAP, _D2), lambda e: (e, 0)),
        out_shape=jax.ShapeDtypeStruct((_E * _CAP, _D2), jnp.int32),
    )(ei, W1, b1.reshape(_E, 1, _DFF), W2, b2.reshape(_E, 1, _D))


def _combine(eo, gidx):
    mesh = plsc.VectorSubcoreMesh(core_axis_name="c", subcore_axis_name="s")

    @functools.partial(
        pl.kernel,
        out_type=jax.ShapeDtypeStruct((_SLOTS, _D2), jnp.int32),
        mesh=mesh,
        scratch_types=[
            pltpu.VMEM((_CHUNK,), jnp.int32),
            pltpu.VMEM((_CHUNK, _D2), jnp.int32),
            pltpu.SemaphoreType.DMA,
        ],
    )
    def k(eo_hbm, idx_hbm, out_hbm, idx_v, rows_v, sem):
        wid = lax.axis_index("s") * _NC + lax.axis_index("c")
        base = wid * _PER_W

        @pl.loop(0, _NCHUNK)
        def _(ci):
            off = base + ci * _CHUNK
            pltpu.sync_copy(idx_hbm.at[pl.ds(off, _CHUNK)], idx_v)
            pltpu.async_copy(eo_hbm.at[idx_v], rows_v, sem).wait()
            pltpu.sync_copy(rows_v, out_hbm.at[pl.ds(off, _CHUNK)])

    return k(eo, gidx)


def _unpack_halves(x32):
    lo = pltpu.unpack_elementwise(
        x32, index=0, packed_dtype=jnp.bfloat16, unpacked_dtype=jnp.float32)
    hi = pltpu.unpack_elementwise(
        x32, index=1, packed_dtype=jnp.bfloat16, unpacked_dtype=jnp.float32)
    return lo, hi


def _head_body(g0_ref, g1_ref, gk0_ref, gk1_ref, wh_ref, o_ref):
    gk0 = gk0_ref[...]
    gk1 = gk1_ref[...]
    lo0, hi0 = _unpack_halves(g0_ref[...])
    lo1, hi1 = _unpack_halves(g1_ref[...])
    moe_lo = jnp.where(gk0 > 0.0, lo0 * gk0, 0.0)
    moe_lo = moe_lo + jnp.where(gk1 > 0.0, lo1 * gk1, 0.0)
    moe_hi = jnp.where(gk0 > 0.0, hi0 * gk0, 0.0)
    moe_hi = moe_hi + jnp.where(gk1 > 0.0, hi1 * gk1, 0.0)
    wh = wh_ref[...]
    out = jnp.dot(moe_lo.astype(jnp.bfloat16),
                  wh[:_D2].astype(jnp.bfloat16),
                  preferred_element_type=jnp.float32)
    out += jnp.dot(moe_hi.astype(jnp.bfloat16),
                   wh[_D2:].astype(jnp.bfloat16),
                   preferred_element_type=jnp.float32)
    o_ref[...] = out


def _head(g, gk0, gk1, W_head):
    return pl.pallas_call(
        _head_body,
        grid=(_T // _TB,),
        in_specs=[
            pl.BlockSpec((_TB, _D2), lambda i: (i, 0)),
            pl.BlockSpec((_TB, _D2), lambda i: (i + _T // _TB, 0)),
            pl.BlockSpec((_TB, 1), lambda i: (i, 0)),
            pl.BlockSpec((_TB, 1), lambda i: (i, 0)),
            pl.BlockSpec((_D, _OUT), lambda i: (0, 0)),
        ],
        out_specs=pl.BlockSpec((_TB, _OUT), lambda i: (i, 0)),
        out_shape=jax.ShapeDtypeStruct((_T, _OUT), jnp.float32),
    )(g, g, gk0, gk1, W_head)


def kernel(x, W_in, b_in, W_router, W1, b1, W2, b2, W_head):
    h, d0, d1, gi0, gi1, gk0, gk1 = _projroute(
        x, W_in, b_in.reshape(1, _D), W_router)
    scat = jnp.concatenate([d0.reshape(_T), d1.reshape(_T)])
    gidx = jnp.concatenate([gi0.reshape(_T), gi1.reshape(_T)])
    ei = _dispatch(h, scat)
    eo = _ffn(ei, W1, b1, W2, b2)
    g = _combine(eo, gidx)
    return _head(g, gk0, gk1, W_head)


# speedup vs baseline: 1.1695x; 1.0184x over previous
import functools

import jax
import jax.numpy as jnp
from jax import lax
from jax.experimental import pallas as pl
from jax.experimental.pallas import tpu as pltpu
from jax.experimental.pallas import tpu_sc as plsc

_T, _D, _E, _K, _DFF, _OUT, _CAP = 2048, 1024, 8, 2, 2048, 1024, 640
_TRASH = _E * _CAP
_EIN_ROWS = _E * _CAP + 8
_TB = 256
_NC, _NS = 2, 16
_NW = _NC * _NS
_SLOTS = _T * _K
_PER_W = _SLOTS // _NW
_CHUNK = 128
_NCHUNK = _PER_W // _CHUNK
_D2 = _D // 2


def _routing_math(lg, d0_ref, d1_ref, gi0_ref, gi1_ref, gk0_ref, gk1_ref):
    lanes = lax.broadcasted_iota(jnp.int32, (_T, _E), 1)
    m0 = jnp.max(lg, axis=1, keepdims=True)
    i0 = jnp.min(jnp.where(lg == m0, lanes, _E), axis=1, keepdims=True)
    masked = jnp.where(lanes == i0, -jnp.inf, lg)
    m1 = jnp.max(masked, axis=1, keepdims=True)
    i1 = jnp.min(jnp.where(masked == m1, lanes, _E), axis=1, keepdims=True)
    e1m = jnp.exp(m1 - m0)
    denom = 1.0 + e1m
    g0 = 1.0 / denom
    g1 = e1m / denom
    oh0 = (lanes == i0).astype(jnp.float32)
    oh1 = (lanes == i1).astype(jnp.float32)
    cnt = (oh0 + oh1).astype(jnp.bfloat16)
    row = lax.broadcasted_iota(jnp.int32, (_T, _T), 0)
    col = lax.broadcasted_iota(jnp.int32, (_T, _T), 1)
    tril = (col < row).astype(jnp.bfloat16)
    csum = jnp.dot(tril, cnt, preferred_element_type=jnp.float32)
    pos0 = jnp.sum(csum * oh0, axis=1, keepdims=True).astype(jnp.int32)
    pos1 = jnp.sum(csum * oh1, axis=1, keepdims=True).astype(jnp.int32)
    keep0 = pos0 < _CAP
    keep1 = pos1 < _CAP
    d0_ref[...] = jnp.where(keep0, i0 * _CAP + pos0, _TRASH)
    d1_ref[...] = jnp.where(keep1, i1 * _CAP + pos1, _TRASH)
    gi0_ref[...] = i0 * _CAP + jnp.minimum(pos0, _CAP - 1)
    gi1_ref[...] = i1 * _CAP + jnp.minimum(pos1, _CAP - 1)
    gk0_ref[...] = jnp.where(keep0, g0, 0.0)
    gk1_ref[...] = jnp.where(keep1, g1, 0.0)


def _projroute_body(x_ref, w_ref, b_ref, wr_ref,
                    h_ref, d0_ref, d1_ref, gi0_ref, gi1_ref, gk0_ref, gk1_ref,
                    lg_acc):
    i = pl.program_id(0)
    h = jnp.dot(x_ref[...], w_ref[...], preferred_element_type=jnp.float32)
    h = h + b_ref[...]
    h_ref[...] = pltpu.pack_elementwise(
        [h[:, :_D2], h[:, _D2:]], packed_dtype=jnp.bfloat16)
    lg_acc[pl.ds(i * _TB, _TB), :] = jnp.dot(
        h, wr_ref[...], preferred_element_type=jnp.float32)

    @pl.when(i == _T // _TB - 1)
    def _():
        _routing_math(lg_acc[...], d0_ref, d1_ref, gi0_ref, gi1_ref,
                      gk0_ref, gk1_ref)


def _projroute(x, W_in, b_in2d, W_router):
    col_i = pl.BlockSpec((_T, 1), lambda i: (0, 0))
    return pl.pallas_call(
        _projroute_body,
        grid=(_T // _TB,),
        in_specs=[
            pl.BlockSpec((_TB, _D), lambda i: (i, 0)),
            pl.BlockSpec((_D, _D), lambda i: (0, 0)),
            pl.BlockSpec((1, _D), lambda i: (0, 0)),
            pl.BlockSpec((_D, _E), lambda i: (0, 0)),
        ],
        out_specs=[pl.BlockSpec((_TB, _D2), lambda i: (i, 0))] + [col_i] * 6,
        out_shape=[jax.ShapeDtypeStruct((_T, _D2), jnp.int32)] + [
            jax.ShapeDtypeStruct((_T, 1), jnp.int32),
            jax.ShapeDtypeStruct((_T, 1), jnp.int32),
            jax.ShapeDtypeStruct((_T, 1), jnp.int32),
            jax.ShapeDtypeStruct((_T, 1), jnp.int32),
            jax.ShapeDtypeStruct((_T, 1), jnp.float32),
            jax.ShapeDtypeStruct((_T, 1), jnp.float32),
        ],
        scratch_shapes=[pltpu.VMEM((_T, _E), jnp.float32)],
    )(x, W_in, b_in2d, W_router)


def _dispatch(h, scat_idx):
    mesh = plsc.VectorSubcoreMesh(core_axis_name="c", subcore_axis_name="s")

    @functools.partial(
        pl.kernel,
        out_type=jax.ShapeDtypeStruct((_EIN_ROWS, _D2), jnp.int32),
        mesh=mesh,
        scratch_types=[
            pltpu.VMEM((_CHUNK,), jnp.int32),
            pltpu.VMEM((_CHUNK, _D2), jnp.int32),
            pltpu.SemaphoreType.DMA,
        ],
    )
    def k(h_hbm, idx_hbm, out_hbm, idx_v, rows_v, sem):
        wid = lax.axis_index("s") * _NC + lax.axis_index("c")
        base = wid * _PER_W

        @pl.loop(0, _NCHUNK)
        def _(ci):
            off = base + ci * _CHUNK
            tok = lax.rem(off, _T)
            pltpu.sync_copy(idx_hbm.at[pl.ds(off, _CHUNK)], idx_v)
            pltpu.sync_copy(h_hbm.at[pl.ds(tok, _CHUNK)], rows_v)
            pltpu.async_copy(rows_v, out_hbm.at[idx_v], sem).wait()

    return k(h, scat_idx)


def _ffn_body(xin_ref, w1_ref, b1_ref, w2_ref, b2_ref, eo_ref):
    x32 = xin_ref[...]
    lo = pltpu.unpack_elementwise(
        x32, index=0, packed_dtype=jnp.bfloat16,
        unpacked_dtype=jnp.float32).astype(jnp.bfloat16)
    hi = pltpu.unpack_elementwise(
        x32, index=1, packed_dtype=jnp.bfloat16,
        unpacked_dtype=jnp.float32).astype(jnp.bfloat16)
    w1 = w1_ref[0]
    hmid = jnp.dot(lo, w1[:_D2].astype(jnp.bfloat16),
                   preferred_element_type=jnp.float32)
    hmid += jnp.dot(hi, w1[_D2:].astype(jnp.bfloat16),
                    preferred_element_type=jnp.float32)
    hmid = jnp.maximum(hmid + b1_ref[0], 0.0).astype(jnp.bfloat16)
    w2 = w2_ref[0].astype(jnp.bfloat16)
    eo = jnp.dot(hmid, w2, preferred_element_type=jnp.float32)
    eo = eo + b2_ref[0]
    eo_ref[...] = pltpu.pack_elementwise(
        [eo[:, :_D2], eo[:, _D2:]], packed_dtype=jnp.bfloat16)


def _ffn(ei, W1, b1, W2, b2):
    return pl.pallas_call(
        _ffn_body,
        grid=(_E,),
        in_specs=[
            pl.BlockSpec((_CAP, _D2), lambda e: (e, 0)),
            pl.BlockSpec((1, _D, _DFF), lambda e: (e, 0, 0)),
            pl.BlockSpec((1, 1, _DFF), lambda e: (e, 0, 0)),
            pl.BlockSpec((1, _DFF, _D), lambda e: (e, 0, 0)),
            pl.BlockSpec((1, 1, _D), lambda e: (e, 0, 0)),
        ],
        out_specs=pl.BlockSpec((_CAP, _D2), lambda e: (e, 0)),
        out_shape=jax.ShapeDtypeStruct((_E * _CAP, _D2), jnp.int32),
    )(ei, W1, b1.reshape(_E, 1, _DFF), W2, b2.reshape(_E, 1, _D))


def _combine(eo, gidx):
    mesh = plsc.VectorSubcoreMesh(core_axis_name="c", subcore_axis_name="s")

    @functools.partial(
        pl.kernel,
        out_type=jax.ShapeDtypeStruct((_SLOTS, _D2), jnp.int32),
        mesh=mesh,
        scratch_types=[
            pltpu.VMEM((_CHUNK,), jnp.int32),
            pltpu.VMEM((_CHUNK, _D2), jnp.int32),
            pltpu.SemaphoreType.DMA,
        ],
    )
    def k(eo_hbm, idx_hbm, out_hbm, idx_v, rows_v, sem):
        wid = lax.axis_index("s") * _NC + lax.axis_index("c")
        base = wid * _PER_W

        @pl.loop(0, _NCHUNK)
        def _(ci):
            off = base + ci * _CHUNK
            pltpu.sync_copy(idx_hbm.at[pl.ds(off, _CHUNK)], idx_v)
            pltpu.async_copy(eo_hbm.at[idx_v], rows_v, sem).wait()
            pltpu.sync_copy(rows_v, out_hbm.at[pl.ds(off, _CHUNK)])

    return k(eo, gidx)


def _unpack_halves(x32):
    lo = pltpu.unpack_elementwise(
        x32, index=0, packed_dtype=jnp.bfloat16, unpacked_dtype=jnp.float32)
    hi = pltpu.unpack_elementwise(
        x32, index=1, packed_dtype=jnp.bfloat16, unpacked_dtype=jnp.float32)
    return lo, hi


def _head_body(g0_ref, g1_ref, gk0_ref, gk1_ref, wh_ref, o_ref):
    gk0 = gk0_ref[...]
    gk1 = gk1_ref[...]
    lo0, hi0 = _unpack_halves(g0_ref[...])
    lo1, hi1 = _unpack_halves(g1_ref[...])
    moe_lo = jnp.where(gk0 > 0.0, lo0 * gk0, 0.0)
    moe_lo = moe_lo + jnp.where(gk1 > 0.0, lo1 * gk1, 0.0)
    moe_hi = jnp.where(gk0 > 0.0, hi0 * gk0, 0.0)
    moe_hi = moe_hi + jnp.where(gk1 > 0.0, hi1 * gk1, 0.0)
    wh = wh_ref[...]
    out = jnp.dot(moe_lo.astype(jnp.bfloat16),
                  wh[:_D2].astype(jnp.bfloat16),
                  preferred_element_type=jnp.float32)
    out += jnp.dot(moe_hi.astype(jnp.bfloat16),
                   wh[_D2:].astype(jnp.bfloat16),
                   preferred_element_type=jnp.float32)
    o_ref[...] = out


def _head(g, gk0, gk1, W_head):
    return pl.pallas_call(
        _head_body,
        grid=(_T // _TB,),
        in_specs=[
            pl.BlockSpec((_TB, _D2), lambda i: (i, 0)),
            pl.BlockSpec((_TB, _D2), lambda i: (i + _T // _TB, 0)),
            pl.BlockSpec((_TB, 1), lambda i: (i, 0)),
            pl.BlockSpec((_TB, 1), lambda i: (i, 0)),
            pl.BlockSpec((_D, _OUT), lambda i: (0, 0)),
        ],
        out_specs=pl.BlockSpec((_TB, _OUT), lambda i: (i, 0)),
        out_shape=jax.ShapeDtypeStruct((_T, _OUT), jnp.float32),
    )(g, g, gk0, gk1, W_head)


def kernel(x, W_in, b_in, W_router, W1, b1, W2, b2, W_head):
    h, d0, d1, gi0, gi1, gk0, gk1 = _projroute(
        x, W_in, b_in.reshape(1, _D), W_router)
    scat = jnp.concatenate([d0.reshape(_T), d1.reshape(_T)])
    gidx = jnp.concatenate([gi0.reshape(_T), gi1.reshape(_T)])
    ei = _dispatch(h, scat)
    eo = _ffn(ei, W1, b1, W2, b2)
    g = _combine(eo, gidx)
    return _head(g, gk0, gk1, W_head)
